# baseline (speedup 1.0000x reference)
import jax
import jax.numpy as jnp
from jax.experimental import pallas as pl
from jax.experimental.pallas import tpu as pltpu
from jax.experimental.pallas import tpu_sc as plsc


_GATHER_WINDOW = 128
_ROW_SPLIT = 2


def _sc_gather(token_emb, idx):
    seq = idx.shape[1]
    dim = token_emb.shape[1]
    mesh = plsc.VectorSubcoreMesh(core_axis_name="core", subcore_axis_name="subcore")
    n_workers = mesh.num_cores * mesh.num_subcores
    rows = seq // n_workers

    @pl.kernel(
        out_type=jax.ShapeDtypeStruct((seq, dim), token_emb.dtype),
        mesh=mesh,
        scratch_types=[
            pltpu.VMEM((1, seq), jnp.int32),
            pltpu.VMEM((rows, dim), token_emb.dtype),
            pltpu.SemaphoreType.DMA,
        ],
    )
    def gather_kernel(emb_hbm, i_hbm, o_hbm, idx_vmem, buf, sem):
        w = (jax.lax.axis_index("core") * mesh.num_subcores
             + jax.lax.axis_index("subcore"))
        pltpu.async_copy(i_hbm, idx_vmem, sem).wait()
        my_idx = idx_vmem.at[0, pl.ds(w * rows, rows)]
        pltpu.async_copy(emb_hbm.at[my_idx], buf, sem).wait()
        pltpu.async_copy(buf, o_hbm.at[pl.ds(w * rows, rows)], sem).wait()

    return gather_kernel(token_emb, idx)


_VT = 1024


def _ln_kernel(h_ref, pos_ref, gamma_ref, beta_ref, hnt_ref):
    hs = h_ref[...] + pos_ref[...]
    mu = jnp.mean(hs, axis=-1, keepdims=True)
    var = jnp.mean((hs - mu) ** 2, axis=-1, keepdims=True)
    hn = (hs - mu) * jax.lax.rsqrt(var + 1e-5)
    hn = hn * gamma_ref[...] + beta_ref[...]
    hnt_ref[...] = hn.astype(jnp.bfloat16).T


def _ln(h, pos, gamma2d, beta2d):
    seq, dim = h.shape
    return pl.pallas_call(
        _ln_kernel,
        grid=(1,),
        in_specs=[
            pl.BlockSpec((seq, dim), lambda j: (0, 0)),
            pl.BlockSpec((seq, dim), lambda j: (0, 0)),
            pl.BlockSpec((1, dim), lambda j: (0, 0)),
            pl.BlockSpec((1, dim), lambda j: (0, 0)),
        ],
        out_specs=pl.BlockSpec((dim, seq), lambda j: (0, 0)),
        out_shape=jax.ShapeDtypeStruct((dim, seq), jnp.bfloat16),
    )(h, pos, gamma2d, beta2d)


def _matmul_kernel(hnt_ref, wt_ref, b_ref, out_ref):
    wt = wt_ref[...].astype(jnp.bfloat16)
    acc = jnp.dot(wt, hnt_ref[...], preferred_element_type=jnp.float32)
    out_ref[...] = acc + b_ref[...]


def _matmul(hnt, Wt, bcol):
    dim, seq = hnt.shape
    vocab = Wt.shape[0]
    grid = (vocab + _VT - 1) // _VT
    return pl.pallas_call(
        _matmul_kernel,
        grid=(grid,),
        in_specs=[
            pl.BlockSpec((dim, seq), lambda j: (0, 0)),
            pl.BlockSpec((_VT, dim), lambda j: (j, 0)),
            pl.BlockSpec((_VT, 1), lambda j: (j, 0)),
        ],
        out_specs=pl.BlockSpec((_VT, seq), lambda j: (j, 0)),
        out_shape=jax.ShapeDtypeStruct((vocab, seq), jnp.float32),
        compiler_params=pltpu.CompilerParams(
            dimension_semantics=("arbitrary",),
        ),
    )(hnt, Wt, bcol)


def _ln_matmul(h, pos, gamma2d, beta2d, W, bcol):
    hnt = _ln(h, pos, gamma2d, beta2d)
    return _matmul(hnt, jnp.swapaxes(W, 0, 1), bcol)


@jax.jit
def kernel(x, token_emb, pos_emb, gamma, beta, W, b):
    batch, seq = x.shape
    dim = token_emb.shape[1]
    idx = x.reshape(1, batch * seq).astype(jnp.int32)
    h = _sc_gather(token_emb, idx)
    logits_t = _ln_matmul(
        h,
        pos_emb,
        gamma.reshape(1, dim),
        beta.reshape(1, dim),
        W,
        b.reshape(-1, 1),
    )
    return jnp.swapaxes(logits_t, 0, 1).reshape(batch, seq, W.shape[1])


# speedup vs baseline: 1.0636x; 1.0636x over previous
import jax
import jax.numpy as jnp
from jax.experimental import pallas as pl
from jax.experimental.pallas import tpu as pltpu
from jax.experimental.pallas import tpu_sc as plsc


_GATHER_WINDOW = 128
_ROW_SPLIT = 2


def _sc_gather(token_emb, idx):
    seq = idx.shape[1]
    dim = token_emb.shape[1]
    mesh = plsc.VectorSubcoreMesh(core_axis_name="core", subcore_axis_name="subcore")
    n_workers = mesh.num_cores * mesh.num_subcores
    rows = seq // n_workers

    @pl.kernel(
        out_type=jax.ShapeDtypeStruct((seq, dim), token_emb.dtype),
        mesh=mesh,
        scratch_types=[
            pltpu.VMEM((1, seq), jnp.int32),
            pltpu.VMEM((rows, dim), token_emb.dtype),
            pltpu.SemaphoreType.DMA,
        ],
    )
    def gather_kernel(emb_hbm, i_hbm, o_hbm, idx_vmem, buf, sem):
        w = (jax.lax.axis_index("core") * mesh.num_subcores
             + jax.lax.axis_index("subcore"))
        pltpu.async_copy(i_hbm, idx_vmem, sem).wait()
        my_idx = idx_vmem.at[0, pl.ds(w * rows, rows)]
        pltpu.async_copy(emb_hbm.at[my_idx], buf, sem).wait()
        pltpu.async_copy(buf, o_hbm.at[pl.ds(w * rows, rows)], sem).wait()

    return gather_kernel(token_emb, idx)


_VT = 2048


def _ln_kernel(h_ref, pos_ref, gamma_ref, beta_ref, hnt_ref):
    hs = h_ref[...] + pos_ref[...]
    mu = jnp.mean(hs, axis=-1, keepdims=True)
    var = jnp.mean((hs - mu) ** 2, axis=-1, keepdims=True)
    hn = (hs - mu) * jax.lax.rsqrt(var + 1e-5)
    hn = hn * gamma_ref[...] + beta_ref[...]
    hnt_ref[...] = hn.astype(jnp.bfloat16).T


def _ln(h, pos, gamma2d, beta2d):
    seq, dim = h.shape
    return pl.pallas_call(
        _ln_kernel,
        grid=(1,),
        in_specs=[
            pl.BlockSpec((seq, dim), lambda j: (0, 0)),
            pl.BlockSpec((seq, dim), lambda j: (0, 0)),
            pl.BlockSpec((1, dim), lambda j: (0, 0)),
            pl.BlockSpec((1, dim), lambda j: (0, 0)),
        ],
        out_specs=pl.BlockSpec((dim, seq), lambda j: (0, 0)),
        out_shape=jax.ShapeDtypeStruct((dim, seq), jnp.bfloat16),
    )(h, pos, gamma2d, beta2d)


def _matmul_kernel(hnt_ref, wt_ref, b_ref, out_ref):
    wt = wt_ref[...].astype(jnp.bfloat16)
    acc = jnp.dot(wt, hnt_ref[...], preferred_element_type=jnp.float32)
    out_ref[...] = acc + b_ref[...]


def _matmul(hnt, Wt, bcol):
    dim, seq = hnt.shape
    vocab = Wt.shape[0]
    grid = (vocab + _VT - 1) // _VT
    return pl.pallas_call(
        _matmul_kernel,
        grid=(grid,),
        in_specs=[
            pl.BlockSpec((dim, seq), lambda j: (0, 0)),
            pl.BlockSpec((_VT, dim), lambda j: (j, 0)),
            pl.BlockSpec((_VT, 1), lambda j: (j, 0)),
        ],
        out_specs=pl.BlockSpec((_VT, seq), lambda j: (j, 0)),
        out_shape=jax.ShapeDtypeStruct((vocab, seq), jnp.float32),
        compiler_params=pltpu.CompilerParams(
            dimension_semantics=("parallel",),
        ),
    )(hnt, Wt, bcol)


def _ln_matmul(h, pos, gamma2d, beta2d, W, bcol):
    hnt = _ln(h, pos, gamma2d, beta2d)
    return _matmul(hnt, jnp.swapaxes(W, 0, 1), bcol)


@jax.jit
def kernel(x, token_emb, pos_emb, gamma, beta, W, b):
    batch, seq = x.shape
    dim = token_emb.shape[1]
    idx = x.reshape(1, batch * seq).astype(jnp.int32)
    h = _sc_gather(token_emb, idx)
    logits_t = _ln_matmul(
        h,
        pos_emb,
        gamma.reshape(1, dim),
        beta.reshape(1, dim),
        W,
        b.reshape(-1, 1),
    )
    return jnp.swapaxes(logits_t, 0, 1).reshape(batch, seq, W.shape[1])
